# baseline (speedup 1.0000x reference)
import jax
import jax.numpy as jnp
from jax.experimental import pallas as pl

_N = 256
_BB_MIN = -2.0
_BB_MAX = 2.0
_VOXEL_SIZE = (_BB_MAX - _BB_MIN) / (_N - 1)
_INV2VS = 1.0 / (2.0 * _VOXEL_SIZE)

_BX = 16
_NUM_BLOCKS = _N // _BX


def _normals_body(c_ref, ph_ref, nh_ref, o_ref):
    c = c_ref[...]
    inv = jnp.float32(_INV2VS)

    xp = jnp.concatenate([c[1:], nh_ref[...]], axis=0)
    xm = jnp.concatenate([ph_ref[...], c[: _BX - 1]], axis=0)
    o_ref[0] = (xp - xm) * inv

    yp_edge = 1.5 * c[:, -1:, :] - 0.5 * c[:, -3:-2, :]
    ym_edge = 1.5 * c[:, 0:1, :] - 0.5 * c[:, 2:3, :]
    yp = jnp.concatenate([c[:, 1:, :], yp_edge], axis=1)
    ym = jnp.concatenate([ym_edge, c[:, : _N - 1, :]], axis=1)
    o_ref[1] = (yp - ym) * inv

    zp_edge = 1.5 * c[:, :, -1:] - 0.5 * c[:, :, -3:-2]
    zm_edge = 1.5 * c[:, :, 0:1] - 0.5 * c[:, :, 2:3]
    zp = jnp.concatenate([c[:, :, 1:], zp_edge], axis=2)
    zm = jnp.concatenate([zm_edge, c[:, :, : _N - 1]], axis=2)
    o_ref[2] = (zp - zm) * inv

    i = pl.program_id(0)

    @pl.when(i == 0)
    def _fix_first():
        o_ref[0, 0] = (c[1] - 1.5 * c[0] + 0.5 * c[2]) * inv

    @pl.when(i == _NUM_BLOCKS - 1)
    def _fix_last():
        o_ref[0, _BX - 1] = (
            1.5 * c[_BX - 1] - c[_BX - 2] - 0.5 * c[_BX - 3]
        ) * inv


def kernel(grid):
    return pl.pallas_call(
        _normals_body,
        grid=(_NUM_BLOCKS,),
        in_specs=[
            pl.BlockSpec((_BX, _N, _N), lambda i: (i, 0, 0)),
            pl.BlockSpec(
                (1, _N, _N), lambda i: (jnp.maximum(i * _BX - 1, 0), 0, 0)
            ),
            pl.BlockSpec(
                (1, _N, _N),
                lambda i: (jnp.minimum(i * _BX + _BX, _N - 1), 0, 0),
            ),
        ],
        out_specs=pl.BlockSpec((3, _BX, _N, _N), lambda i: (0, i, 0, 0)),
        out_shape=jax.ShapeDtypeStruct((3, _N, _N, _N), jnp.float32),
    )(grid, grid, grid)


# speedup vs baseline: 1.0039x; 1.0039x over previous
import jax
import jax.numpy as jnp
from jax.experimental import pallas as pl

_N = 256
_BB_MIN = -2.0
_BB_MAX = 2.0
_VOXEL_SIZE = (_BB_MAX - _BB_MIN) / (_N - 1)
_INV2VS = 1.0 / (2.0 * _VOXEL_SIZE)

_BX = 16
_NUM_BLOCKS = _N // _BX


def _normals_body(c_ref, ph_ref, nh_ref, o_ref):
    inv = jnp.float32(_INV2VS)
    c = c_ref[...] * inv

    o_ref[0, 1 : _BX - 1] = c[2:] - c[: _BX - 2]
    o_ref[0, 0:1] = c[1:2] - ph_ref[...] * inv
    o_ref[0, _BX - 1 : _BX] = nh_ref[...] * inv - c[_BX - 2 : _BX - 1]

    yp_edge = 1.5 * c[:, -1:, :] - 0.5 * c[:, -3:-2, :]
    ym_edge = 1.5 * c[:, 0:1, :] - 0.5 * c[:, 2:3, :]
    yp = jnp.concatenate([c[:, 1:, :], yp_edge], axis=1)
    ym = jnp.concatenate([ym_edge, c[:, : _N - 1, :]], axis=1)
    o_ref[1] = yp - ym

    zp_edge = 1.5 * c[:, :, -1:] - 0.5 * c[:, :, -3:-2]
    zm_edge = 1.5 * c[:, :, 0:1] - 0.5 * c[:, :, 2:3]
    zp = jnp.concatenate([c[:, :, 1:], zp_edge], axis=2)
    zm = jnp.concatenate([zm_edge, c[:, :, : _N - 1]], axis=2)
    o_ref[2] = zp - zm

    i = pl.program_id(0)

    @pl.when(i == 0)
    def _fix_first():
        o_ref[0, 0] = c[1] - 1.5 * c[0] + 0.5 * c[2]

    @pl.when(i == _NUM_BLOCKS - 1)
    def _fix_last():
        o_ref[0, _BX - 1] = (
            1.5 * c[_BX - 1] - c[_BX - 2] - 0.5 * c[_BX - 3]
        )


def kernel(grid):
    return pl.pallas_call(
        _normals_body,
        grid=(_NUM_BLOCKS,),
        in_specs=[
            pl.BlockSpec((_BX, _N, _N), lambda i: (i, 0, 0)),
            pl.BlockSpec(
                (1, _N, _N), lambda i: (jnp.maximum(i * _BX - 1, 0), 0, 0)
            ),
            pl.BlockSpec(
                (1, _N, _N),
                lambda i: (jnp.minimum(i * _BX + _BX, _N - 1), 0, 0),
            ),
        ],
        out_specs=pl.BlockSpec((3, _BX, _N, _N), lambda i: (0, i, 0, 0)),
        out_shape=jax.ShapeDtypeStruct((3, _N, _N, _N), jnp.float32),
    )(grid, grid, grid)


# speedup vs baseline: 1.0203x; 1.0164x over previous
import jax
import jax.numpy as jnp
from jax.experimental import pallas as pl
from jax.experimental.pallas import tpu as pltpu

_N = 256
_BB_MIN = -2.0
_BB_MAX = 2.0
_VOXEL_SIZE = (_BB_MAX - _BB_MIN) / (_N - 1)
_INV2VS = 1.0 / (2.0 * _VOXEL_SIZE)

_BX = 16
_NUM_BLOCKS = _N // _BX


def _normals_body(c_ref, nh_ref, o_ref, carry_ref):
    inv = jnp.float32(_INV2VS)
    c = c_ref[...] * inv

    o_ref[0, 1 : _BX - 1] = c[2:] - c[: _BX - 2]
    o_ref[0, 0:1] = c[1:2] - carry_ref[...]
    o_ref[0, _BX - 1 : _BX] = nh_ref[...] * inv - c[_BX - 2 : _BX - 1]
    carry_ref[...] = c[_BX - 1 : _BX]

    yp_edge = 1.5 * c[:, -1:, :] - 0.5 * c[:, -3:-2, :]
    ym_edge = 1.5 * c[:, 0:1, :] - 0.5 * c[:, 2:3, :]
    yp = jnp.concatenate([c[:, 1:, :], yp_edge], axis=1)
    ym = jnp.concatenate([ym_edge, c[:, : _N - 1, :]], axis=1)
    o_ref[1] = yp - ym

    zp_edge = 1.5 * c[:, :, -1:] - 0.5 * c[:, :, -3:-2]
    zm_edge = 1.5 * c[:, :, 0:1] - 0.5 * c[:, :, 2:3]
    zp = jnp.concatenate([c[:, :, 1:], zp_edge], axis=2)
    zm = jnp.concatenate([zm_edge, c[:, :, : _N - 1]], axis=2)
    o_ref[2] = zp - zm

    i = pl.program_id(0)

    @pl.when(i == 0)
    def _fix_first():
        o_ref[0, 0] = c[1] - 1.5 * c[0] + 0.5 * c[2]

    @pl.when(i == _NUM_BLOCKS - 1)
    def _fix_last():
        o_ref[0, _BX - 1] = (
            1.5 * c[_BX - 1] - c[_BX - 2] - 0.5 * c[_BX - 3]
        )


def kernel(grid):
    return pl.pallas_call(
        _normals_body,
        grid=(_NUM_BLOCKS,),
        in_specs=[
            pl.BlockSpec((_BX, _N, _N), lambda i: (i, 0, 0)),
            pl.BlockSpec(
                (1, _N, _N),
                lambda i: (jnp.minimum(i * _BX + _BX, _N - 1), 0, 0),
            ),
        ],
        out_specs=pl.BlockSpec((3, _BX, _N, _N), lambda i: (0, i, 0, 0)),
        out_shape=jax.ShapeDtypeStruct((3, _N, _N, _N), jnp.float32),
        scratch_shapes=[pltpu.VMEM((1, _N, _N), jnp.float32)],
    )(grid, grid)
